# baseline (speedup 1.0000x reference)
import numpy as np
import jax
import jax.numpy as jnp
from jax import lax
from jax.experimental import pallas as pl
from jax.experimental.pallas import tpu as pltpu
from jax.experimental.pallas import tpu_sc as plsc

B, T, C, H, E, TOPK = 4, 1024, 384, 6, 8, 2
DH = C // H
CAP = 1024
N = B * T
BLK = 256
NBLK = N // BLK
NSC = 16
ND = E * CAP
F4 = 4 * C

_perm = np.arange(C).reshape(H, DH)
_perm = np.concatenate([_perm[:, DH // 2:], _perm[:, :DH // 2]], axis=1).reshape(-1)
_IND = np.zeros((C, 128), np.float32)
_INDT = np.zeros((128, C), np.float32)
for _h in range(H):
    _IND[_h * DH:(_h + 1) * DH, _h] = 1.0 / DH
    _INDT[_h, _h * DH:(_h + 1) * DH] = 1.0


def _dot(a, b):
    return lax.dot_general(a, b, (((1,), (0,)), ((), ())),
                           preferred_element_type=jnp.float32)


def _qkv_body(x_ref, ln1_ref, wq_ref, wqp_ref, wk_ref, wkp_ref, wv_ref,
              ind_ref, indt_ref, cos_ref, sin_ref, q_ref, k_ref, v_ref):
    xb = x_ref[...]
    ms = jnp.mean(xb * xb, axis=1, keepdims=True)
    hb = xb * lax.rsqrt(ms + 1e-6) * ln1_ref[...]
    qa = _dot(hb, wq_ref[...])
    qb = _dot(hb, wqp_ref[...])
    ka = _dot(hb, wk_ref[...])
    kb = _dot(hb, wkp_ref[...])
    cosb = cos_ref[...]
    sinb = sin_ref[...]

    def headr(sa):
        m = _dot(sa * sa, ind_ref[...])
        return lax.rsqrt(_dot(m, indt_ref[...]) + 1e-6)

    q_ref[...] = (qa * cosb + qb * sinb) * headr(qa)
    k_ref[...] = (ka * cosb + kb * sinb) * headr(ka)
    v_ref[...] = _dot(hb, wv_ref[...])


def _attn_body(q_ref, k_ref, v_ref, o_ref):
    q = q_ref[0]
    k = k_ref[0]
    v = v_ref[0]
    s = lax.dot_general(q, k, (((1,), (1,)), ((), ())),
                        preferred_element_type=jnp.float32) * 0.125
    ri = lax.broadcasted_iota(jnp.int32, (T, T), 0)
    ci = lax.broadcasted_iota(jnp.int32, (T, T), 1)
    s = jnp.where(ri >= ci, s, -1e9)
    m = jnp.max(s, axis=1, keepdims=True)
    p = jnp.exp(s - m)
    a = p / jnp.sum(p, axis=1, keepdims=True)
    o_ref[0] = lax.dot_general(a, v, (((1,), (0,)), ((), ())),
                               preferred_element_type=jnp.float32)


def _proj_router_body(o_ref, x_ref, wc_ref, cpb_ref, ln2_ref, rw_ref, rb_ref,
                      nw_ref, nb_ref, eps_ref, x1_ref, as_ref, wg_ref):
    x1 = _dot(o_ref[...], wc_ref[...]) + cpb_ref[...] + x_ref[...]
    x1_ref[...] = x1
    ms = jnp.mean(x1 * x1, axis=1, keepdims=True)
    h2 = x1 * lax.rsqrt(ms + 1e-6) * ln2_ref[...]
    lg = _dot(h2, rw_ref[...]) + rb_ref[...]
    nl = _dot(h2, nw_ref[...]) + nb_ref[...]
    sp = jnp.maximum(nl, 0.0) + jnp.log(1.0 + jnp.exp(-jnp.abs(nl)))
    nz = lg + eps_ref[...] * sp
    cid = lax.broadcasted_iota(jnp.int32, (BLK, 128), 1)
    nz = jnp.where(cid < E, nz, -1e30)
    v0 = jnp.max(nz, axis=1, keepdims=True)
    a0 = jnp.min(jnp.where(nz == v0, cid, 128), axis=1, keepdims=True)
    v1 = jnp.max(jnp.where(cid == a0, -1e30, nz), axis=1, keepdims=True)
    w = 1.0 / (1.0 + jnp.exp(v1 - v0))
    as_ref[...] = jnp.broadcast_to(a0, (BLK, 128))
    wg_ref[...] = jnp.broadcast_to(w, (BLK, 128))


def _ffn_body(xd_ref, w1_ref, b1_ref, w2_ref, b2_ref, eo_ref):
    mid = _dot(xd_ref[...], w1_ref[0]) + b1_ref[0]
    mid = jnp.square(jnp.maximum(mid, 0.0))
    eo_ref[...] = _dot(mid, w2_ref[0]) + b2_ref[0]


def _hist_body(assign_hbm, cnt_hbm, a_v, cnt_v):
    cid = lax.axis_index("c")
    sid = lax.axis_index("s")
    lanes = lax.iota(jnp.int32, 16)

    @pl.when(cid == 0)
    def _go():
        base = sid * 256
        pltpu.sync_copy(assign_hbm.at[pl.ds(base, 256)], a_v)
        cnt = jnp.zeros((16,), jnp.int32)
        for j in range(16):
            a = a_v[pl.ds(j * 16, 16)]
            for e in range(E):
                pe = plsc.cumsum(jnp.where(a == e, 1, 0))
                cnt = cnt + jnp.where(lanes == e, pe[15], 0)
        cnt_v[...] = cnt
        pltpu.sync_copy(cnt_v, cnt_hbm.at[sid])


def _rank_body(assign_hbm, wgt_hbm, cnt_hbm, slotg_hbm, slotc_hbm, wv_hbm,
               a_v, w_v, allcnt_v, slotg_v, slotc_v):
    cid = lax.axis_index("c")
    sid = lax.axis_index("s")
    lanes = lax.iota(jnp.int32, 16)
    zeros16 = jnp.zeros((16,), jnp.int32)

    @pl.when(cid == 0)
    def _go():
        base = sid * 256
        pltpu.sync_copy(assign_hbm.at[pl.ds(base, 256)], a_v)
        pltpu.sync_copy(wgt_hbm.at[pl.ds(base, 256)], w_v)
        pltpu.sync_copy(cnt_hbm, allcnt_v)
        sidv = jnp.full((16,), sid, jnp.int32)
        rc = zeros16
        for w in range(NSC):
            pred = jnp.full((16,), w, jnp.int32) < sidv
            rc = rc + jnp.where(pred, allcnt_v[w], 0)
        for j in range(16):
            a = a_v[pl.ds(j * 16, 16)]
            wr = zeros16
            tot = zeros16
            basec = zeros16
            for e in range(E):
                m = a == e
                pe = plsc.cumsum(jnp.where(m, 1, 0))
                wr = jnp.where(m, pe - 1, wr)
                tot = tot + jnp.where(lanes == e, pe[15], 0)
                basec = jnp.where(m, rc[e], basec)
            grank = basec + wr
            rc = rc + tot
            slot = a * CAP + grank
            valid = grank < CAP
            slotg_v[pl.ds(j * 16, 16)] = jnp.where(valid, slot, ND)
            slotc_v[pl.ds(j * 16, 16)] = jnp.where(valid, slot, 0)
            w_v[pl.ds(j * 16, 16)] = jnp.where(valid, w_v[pl.ds(j * 16, 16)],
                                               0.0)
        pltpu.sync_copy(slotg_v, slotg_hbm.at[pl.ds(base, 256)])
        pltpu.sync_copy(slotc_v, slotc_hbm.at[pl.ds(base, 256)])
        pltpu.sync_copy(w_v, wv_hbm.at[pl.ds(base, 256)])


def _sctc_body(slotg_ref, gf_ref):
    cb = pl.program_id(0)
    tb = pl.program_id(1)
    sl = slotg_ref[0]
    colid = lax.broadcasted_iota(jnp.int32, (256, 1024), 1) + cb * 1024
    oh = (sl == colid).astype(jnp.float32)
    tok = (lax.broadcasted_iota(jnp.int32, (256, 1), 0)
           + 256 * tb).astype(jnp.float32)
    part = lax.dot_general(tok, oh, (((0,), (0,)), ((), ())),
                           precision=lax.Precision.HIGHEST,
                           preferred_element_type=jnp.float32)

    @pl.when(tb == 0)
    def _init():
        gf_ref[0] = part

    @pl.when(tb != 0)
    def _acc():
        gf_ref[0] = gf_ref[0] + part


def _disp_body(gidx_hbm, x1_hbm, xd_hbm, idx_v, rows_v, sem):
    wid = lax.axis_index("s") * 2 + lax.axis_index("c")
    base = wid * 256
    for r in range(2):
        pltpu.sync_copy(gidx_hbm.at[pl.ds(base + r * 128, 128)], idx_v.at[r])
        pltpu.async_copy(x1_hbm.at[idx_v.at[r]],
                         rows_v.at[pl.ds(r * 128, 128)], sem).wait()
    pltpu.sync_copy(rows_v, xd_hbm.at[pl.ds(base, 256)])


def _comb_body(slotc_hbm, eo_hbm, eg_hbm, idx_v, eo_v, sem):
    wid = lax.axis_index("s") * 2 + lax.axis_index("c")
    base = wid * 128
    pltpu.sync_copy(slotc_hbm.at[pl.ds(base, 128)], idx_v)
    pltpu.async_copy(eo_hbm.at[idx_v], eo_v, sem).wait()
    pltpu.sync_copy(eo_v, eg_hbm.at[pl.ds(base, 128)])


def _wsum_body(x1_ref, wv_ref, eg_ref, out_ref):
    out_ref[...] = x1_ref[...] + wv_ref[0] * eg_ref[...]


def kernel(x, ln1_w, qkv_w, c_proj_w, c_proj_b, lamb, ln2_w, route_w, route_b,
           noise_w, noise_b, w1, b1, w2, b2):
    f32 = jnp.float32
    xf = x.reshape(N, C)
    Wq, Wk, Wv = qkv_w[0].T, qkv_w[1].T, qkv_w[2].T
    WqP, WkP = Wq[:, _perm], Wk[:, _perm]
    steps = DH // 4
    inv = (1.0 / 1024.0) ** jnp.linspace(0.0, 1.0, steps)
    inv = jnp.concatenate([inv, jnp.zeros((steps,), f32)])
    th = jnp.arange(T, dtype=f32)[:, None] * inv[None, :]
    cos, sin = jnp.cos(th), jnp.sin(th)
    cosF = jnp.tile(jnp.concatenate([cos, cos], axis=1), (1, H))
    sinF = jnp.tile(jnp.concatenate([sin, -sin], axis=1), (1, H))

    rowspec = pl.BlockSpec((BLK, C), lambda i: (i, 0))
    fullspec = pl.BlockSpec((C, C), lambda i: (0, 0))
    vecspec = pl.BlockSpec((1, C), lambda i: (0, 0))
    cos_spec = pl.BlockSpec((BLK, C), lambda i: (lax.rem(i, T // BLK), 0))
    q, k, v = pl.pallas_call(
        _qkv_body,
        grid=(NBLK,),
        in_specs=[rowspec, vecspec, fullspec, fullspec, fullspec, fullspec,
                  fullspec, pl.BlockSpec((C, 128), lambda i: (0, 0)),
                  pl.BlockSpec((128, C), lambda i: (0, 0)), cos_spec, cos_spec],
        out_specs=[rowspec, rowspec, rowspec],
        out_shape=[jax.ShapeDtypeStruct((N, C), f32)] * 3,
    )(xf, ln1_w.reshape(1, C), Wq, WqP, Wk, WkP, Wv, _IND, _INDT, cosF, sinF)

    def heads(t):
        return t.reshape(B, T, H, DH).transpose(0, 2, 1, 3).reshape(B * H, T, DH)

    hspec = pl.BlockSpec((1, T, DH), lambda i: (i, 0, 0))
    o3 = pl.pallas_call(
        _attn_body,
        grid=(B * H,),
        in_specs=[hspec, hspec, hspec],
        out_specs=hspec,
        out_shape=jax.ShapeDtypeStruct((B * H, T, DH), f32),
    )(heads(q), heads(k), heads(v))
    o = o3.reshape(B, H, T, DH).transpose(0, 2, 1, 3).reshape(N, C)

    RWr = jnp.zeros((C, 128), f32).at[:, :E].set(route_w.T)
    RWn = jnp.zeros((C, 128), f32).at[:, :E].set(noise_w.T)
    RBr = jnp.zeros((1, 128), f32).at[0, :E].set(route_b)
    RBn = jnp.zeros((1, 128), f32).at[0, :E].set(noise_b)
    eps = jax.random.normal(jax.random.key(42), (N, E), f32)
    epsP = jnp.zeros((N, 128), f32).at[:, :E].set(eps)
    pad128 = pl.BlockSpec((C, 128), lambda i: (0, 0))
    vec128 = pl.BlockSpec((1, 128), lambda i: (0, 0))
    row128 = pl.BlockSpec((BLK, 128), lambda i: (i, 0))
    x1, as128, wg128 = pl.pallas_call(
        _proj_router_body,
        grid=(NBLK,),
        in_specs=[rowspec, rowspec, fullspec, vecspec, vecspec,
                  pad128, vec128, pad128, vec128, row128],
        out_specs=[rowspec, row128, row128],
        out_shape=[jax.ShapeDtypeStruct((N, C), f32),
                   jax.ShapeDtypeStruct((N, 128), jnp.int32),
                   jax.ShapeDtypeStruct((N, 128), f32)],
    )(o, xf, c_proj_w.T, c_proj_b.reshape(1, C), ln2_w.reshape(1, C),
      RWr, RBr, RWn, RBn, epsP)
    assign = as128[:, 0]
    wgt = wg128[:, 0]

    mesh2 = plsc.VectorSubcoreMesh(core_axis_name="c", subcore_axis_name="s")
    hist = pl.kernel(
        _hist_body,
        out_type=jax.ShapeDtypeStruct((NSC, 16), jnp.int32),
        mesh=mesh2,
        scratch_types=[pltpu.VMEM((256,), jnp.int32),
                       pltpu.VMEM((16,), jnp.int32)],
        compiler_params=pltpu.CompilerParams(needs_layout_passes=False),
    )
    cnts = hist(assign)
    rank = pl.kernel(
        _rank_body,
        out_type=[jax.ShapeDtypeStruct((N,), jnp.int32),
                  jax.ShapeDtypeStruct((N,), jnp.int32),
                  jax.ShapeDtypeStruct((N,), f32)],
        mesh=mesh2,
        scratch_types=[pltpu.VMEM((256,), jnp.int32),
                       pltpu.VMEM((256,), f32),
                       pltpu.VMEM((NSC, 16), jnp.int32),
                       pltpu.VMEM((256,), jnp.int32),
                       pltpu.VMEM((256,), jnp.int32)],
        compiler_params=pltpu.CompilerParams(needs_layout_passes=False),
    )
    slotg, slotc, wv = rank(assign, wgt, cnts)

    gf = pl.pallas_call(
        _sctc_body,
        grid=(9, 16),
        in_specs=[pl.BlockSpec((1, 256, 1), lambda cb, tb: (tb, 0, 0))],
        out_specs=pl.BlockSpec((1, 1, 1024), lambda cb, tb: (cb, 0, 0)),
        out_shape=jax.ShapeDtypeStruct((9, 1, 1024), f32),
    )(slotg.reshape(16, 256, 1))
    gidx = gf.reshape(-1)[:ND].astype(jnp.int32)

    disp = pl.kernel(
        _disp_body,
        out_type=jax.ShapeDtypeStruct((ND, C), f32),
        mesh=mesh2,
        scratch_types=[pltpu.VMEM((2, 128), jnp.int32),
                       pltpu.VMEM((256, C), f32),
                       pltpu.SemaphoreType.DMA],
    )
    xd = disp(gidx, x1)

    espec = pl.BlockSpec((BLK, C), lambda e, m: (e * (CAP // BLK) + m, 0))
    eo = pl.pallas_call(
        _ffn_body,
        grid=(E, CAP // BLK),
        in_specs=[espec,
                  pl.BlockSpec((1, C, F4), lambda e, m: (e, 0, 0)),
                  pl.BlockSpec((1, 1, F4), lambda e, m: (e, 0, 0)),
                  pl.BlockSpec((1, F4, C), lambda e, m: (e, 0, 0)),
                  pl.BlockSpec((1, 1, C), lambda e, m: (e, 0, 0))],
        out_specs=espec,
        out_shape=jax.ShapeDtypeStruct((ND, C), f32),
    )(xd, jnp.swapaxes(w1, 1, 2), b1.reshape(E, 1, F4), jnp.swapaxes(w2, 1, 2),
      b2.reshape(E, 1, C))

    comb = pl.kernel(
        _comb_body,
        out_type=jax.ShapeDtypeStruct((N, C), f32),
        mesh=mesh2,
        scratch_types=[pltpu.VMEM((128,), jnp.int32),
                       pltpu.VMEM((128, C), f32),
                       pltpu.SemaphoreType.DMA],
    )
    eg = comb(slotc, eo)
    out = pl.pallas_call(
        _wsum_body,
        grid=(NBLK,),
        in_specs=[rowspec, pl.BlockSpec((1, BLK, 1), lambda i: (i, 0, 0)),
                  rowspec],
        out_specs=rowspec,
        out_shape=jax.ShapeDtypeStruct((N, C), f32),
    )(x1, wv.reshape(NBLK, BLK, 1), eg)
    return out.reshape(B, T, C)


# speedup vs baseline: 1.0069x; 1.0069x over previous
import numpy as np
import jax
import jax.numpy as jnp
from jax import lax
from jax.experimental import pallas as pl
from jax.experimental.pallas import tpu as pltpu
from jax.experimental.pallas import tpu_sc as plsc

B, T, C, H, E, TOPK = 4, 1024, 384, 6, 8, 2
DH = C // H
CAP = 1024
N = B * T
BLK = 256
NBLK = N // BLK
NSC = 16
ND = E * CAP
F4 = 4 * C

_perm = np.arange(C).reshape(H, DH)
_perm = np.concatenate([_perm[:, DH // 2:], _perm[:, :DH // 2]], axis=1).reshape(-1)
_IND = np.zeros((C, 128), np.float32)
_INDT = np.zeros((128, C), np.float32)
for _h in range(H):
    _IND[_h * DH:(_h + 1) * DH, _h] = 1.0 / DH
    _INDT[_h, _h * DH:(_h + 1) * DH] = 1.0


def _dot(a, b):
    return lax.dot_general(a, b, (((1,), (0,)), ((), ())),
                           preferred_element_type=jnp.float32)


def _qkv_body(x_ref, ln1_ref, wq_ref, wqp_ref, wk_ref, wkp_ref, wv_ref,
              ind_ref, indt_ref, cos_ref, sin_ref, q_ref, k_ref, v_ref):
    xb = x_ref[...]
    ms = jnp.mean(xb * xb, axis=1, keepdims=True)
    hb = xb * lax.rsqrt(ms + 1e-6) * ln1_ref[...]
    qa = _dot(hb, wq_ref[...])
    qb = _dot(hb, wqp_ref[...])
    ka = _dot(hb, wk_ref[...])
    kb = _dot(hb, wkp_ref[...])
    cosb = cos_ref[...]
    sinb = sin_ref[...]

    def headr(sa):
        m = _dot(sa * sa, ind_ref[...])
        return lax.rsqrt(_dot(m, indt_ref[...]) + 1e-6)

    q_ref[...] = (qa * cosb + qb * sinb) * headr(qa)
    k_ref[...] = (ka * cosb + kb * sinb) * headr(ka)
    v_ref[...] = _dot(hb, wv_ref[...])


def _attn_body(q_ref, k_ref, v_ref, o_ref):
    q = q_ref[0]
    k = k_ref[0]
    v = v_ref[0]
    s = lax.dot_general(q, k, (((1,), (1,)), ((), ())),
                        preferred_element_type=jnp.float32) * 0.125
    ri = lax.broadcasted_iota(jnp.int32, (T, T), 0)
    ci = lax.broadcasted_iota(jnp.int32, (T, T), 1)
    s = jnp.where(ri >= ci, s, -1e9)
    m = jnp.max(s, axis=1, keepdims=True)
    p = jnp.exp(s - m)
    a = p / jnp.sum(p, axis=1, keepdims=True)
    o_ref[0] = lax.dot_general(a, v, (((1,), (0,)), ((), ())),
                               preferred_element_type=jnp.float32)


def _proj_router_body(o_ref, x_ref, wc_ref, cpb_ref, ln2_ref, rw_ref, rb_ref,
                      nw_ref, nb_ref, eps_ref, x1_ref, as_ref, wg_ref):
    x1 = _dot(o_ref[...], wc_ref[...]) + cpb_ref[...] + x_ref[...]
    x1_ref[...] = x1
    ms = jnp.mean(x1 * x1, axis=1, keepdims=True)
    h2 = x1 * lax.rsqrt(ms + 1e-6) * ln2_ref[...]
    lg = _dot(h2, rw_ref[...]) + rb_ref[...]
    nl = _dot(h2, nw_ref[...]) + nb_ref[...]
    sp = jnp.maximum(nl, 0.0) + jnp.log(1.0 + jnp.exp(-jnp.abs(nl)))
    nz = lg + eps_ref[...] * sp
    cid = lax.broadcasted_iota(jnp.int32, (BLK, 128), 1)
    nz = jnp.where(cid < E, nz, -1e30)
    v0 = jnp.max(nz, axis=1, keepdims=True)
    a0 = jnp.min(jnp.where(nz == v0, cid, 128), axis=1, keepdims=True)
    v1 = jnp.max(jnp.where(cid == a0, -1e30, nz), axis=1, keepdims=True)
    w = 1.0 / (1.0 + jnp.exp(v1 - v0))
    as_ref[...] = jnp.broadcast_to(a0, (BLK, 128))
    wg_ref[...] = jnp.broadcast_to(w, (BLK, 128))


def _ffn_body(xd_ref, w1_ref, b1_ref, w2_ref, b2_ref, eo_ref):
    mid = _dot(xd_ref[...], w1_ref[0]) + b1_ref[0]
    mid = jnp.square(jnp.maximum(mid, 0.0))
    eo_ref[...] = _dot(mid, w2_ref[0]) + b2_ref[0]


def _hist_body(assign_hbm, cnt_hbm, a_v, cnt_v):
    cid = lax.axis_index("c")
    sid = lax.axis_index("s")
    lanes = lax.iota(jnp.int32, 16)

    @pl.when(cid == 0)
    def _go():
        base = sid * 256
        pltpu.sync_copy(assign_hbm.at[pl.ds(base, 256)], a_v)
        cnt = jnp.zeros((16,), jnp.int32)
        for j in range(16):
            a = a_v[pl.ds(j * 16, 16)]
            for e in range(E):
                pe = plsc.cumsum(jnp.where(a == e, 1, 0))
                cnt = cnt + jnp.where(lanes == e, pe[15], 0)
        cnt_v[...] = cnt
        pltpu.sync_copy(cnt_v, cnt_hbm.at[sid])


def _rank_body(assign_hbm, wgt_hbm, cnt_hbm, slotg_hbm, slotc_hbm, wv_hbm,
               a_v, w_v, allcnt_v, slotg_v, slotc_v):
    cid = lax.axis_index("c")
    sid = lax.axis_index("s")
    lanes = lax.iota(jnp.int32, 16)
    zeros16 = jnp.zeros((16,), jnp.int32)

    @pl.when(cid == 0)
    def _go():
        base = sid * 256
        pltpu.sync_copy(assign_hbm.at[pl.ds(base, 256)], a_v)
        pltpu.sync_copy(wgt_hbm.at[pl.ds(base, 256)], w_v)
        pltpu.sync_copy(cnt_hbm, allcnt_v)
        sidv = jnp.full((16,), sid, jnp.int32)
        rc = zeros16
        for w in range(NSC):
            pred = jnp.full((16,), w, jnp.int32) < sidv
            rc = rc + jnp.where(pred, allcnt_v[w], 0)
        for j in range(16):
            a = a_v[pl.ds(j * 16, 16)]
            wr = zeros16
            tot = zeros16
            basec = zeros16
            for e in range(E):
                m = a == e
                pe = plsc.cumsum(jnp.where(m, 1, 0))
                wr = jnp.where(m, pe - 1, wr)
                tot = tot + jnp.where(lanes == e, pe[15], 0)
                basec = jnp.where(m, rc[e], basec)
            grank = basec + wr
            rc = rc + tot
            slot = a * CAP + grank
            valid = grank < CAP
            slotg_v[pl.ds(j * 16, 16)] = jnp.where(valid, slot, ND)
            slotc_v[pl.ds(j * 16, 16)] = jnp.where(valid, slot, 0)
            w_v[pl.ds(j * 16, 16)] = jnp.where(valid, w_v[pl.ds(j * 16, 16)],
                                               0.0)
        pltpu.sync_copy(slotg_v, slotg_hbm.at[pl.ds(base, 256)])
        pltpu.sync_copy(slotc_v, slotc_hbm.at[pl.ds(base, 256)])
        pltpu.sync_copy(w_v, wv_hbm.at[pl.ds(base, 256)])


def _sctc_body(slotg_ref, gf_ref):
    cb = pl.program_id(0)
    tb = pl.program_id(1)
    sl = slotg_ref[0]
    colid = lax.broadcasted_iota(jnp.int32, (256, 1024), 1) + cb * 1024
    oh = (sl == colid).astype(jnp.float32)
    tok = (lax.broadcasted_iota(jnp.int32, (256, 1), 0)
           + 256 * tb).astype(jnp.float32)
    part = lax.dot_general(tok, oh, (((0,), (0,)), ((), ())),
                           precision=lax.Precision.HIGHEST,
                           preferred_element_type=jnp.float32)

    @pl.when(tb == 0)
    def _init():
        gf_ref[0] = part

    @pl.when(tb != 0)
    def _acc():
        gf_ref[0] = gf_ref[0] + part


def _disp_body(gidx_hbm, x1_hbm, xd_hbm, idx_a, idx_b, rows_a, rows_b, sem):
    wid = lax.axis_index("s") * 2 + lax.axis_index("c")
    base = wid * 256
    pltpu.sync_copy(gidx_hbm.at[pl.ds(base, 128)], idx_a)
    pltpu.sync_copy(gidx_hbm.at[pl.ds(base + 128, 128)], idx_b)
    cp_a = pltpu.async_copy(x1_hbm.at[idx_a], rows_a, sem)
    cp_b = pltpu.async_copy(x1_hbm.at[idx_b], rows_b, sem)
    cp_a.wait()
    cp_b.wait()
    pltpu.sync_copy(rows_a, xd_hbm.at[pl.ds(base, 128)])
    pltpu.sync_copy(rows_b, xd_hbm.at[pl.ds(base + 128, 128)])


def _comb_body(slotc_hbm, eo_hbm, eg_hbm, idx_v, eo_v, sem):
    wid = lax.axis_index("s") * 2 + lax.axis_index("c")
    base = wid * 128
    pltpu.sync_copy(slotc_hbm.at[pl.ds(base, 128)], idx_v)
    pltpu.async_copy(eo_hbm.at[idx_v], eo_v, sem).wait()
    pltpu.sync_copy(eo_v, eg_hbm.at[pl.ds(base, 128)])


def _wsum_body(x1_ref, wv_ref, eg_ref, out_ref):
    out_ref[...] = x1_ref[...] + wv_ref[0] * eg_ref[...]


def kernel(x, ln1_w, qkv_w, c_proj_w, c_proj_b, lamb, ln2_w, route_w, route_b,
           noise_w, noise_b, w1, b1, w2, b2):
    f32 = jnp.float32
    xf = x.reshape(N, C)
    Wq, Wk, Wv = qkv_w[0].T, qkv_w[1].T, qkv_w[2].T
    WqP, WkP = Wq[:, _perm], Wk[:, _perm]
    steps = DH // 4
    inv = (1.0 / 1024.0) ** jnp.linspace(0.0, 1.0, steps)
    inv = jnp.concatenate([inv, jnp.zeros((steps,), f32)])
    th = jnp.arange(T, dtype=f32)[:, None] * inv[None, :]
    cos, sin = jnp.cos(th), jnp.sin(th)
    cosF = jnp.tile(jnp.concatenate([cos, cos], axis=1), (1, H))
    sinF = jnp.tile(jnp.concatenate([sin, -sin], axis=1), (1, H))

    rowspec = pl.BlockSpec((BLK, C), lambda i: (i, 0))
    fullspec = pl.BlockSpec((C, C), lambda i: (0, 0))
    vecspec = pl.BlockSpec((1, C), lambda i: (0, 0))
    cos_spec = pl.BlockSpec((BLK, C), lambda i: (lax.rem(i, T // BLK), 0))
    q, k, v = pl.pallas_call(
        _qkv_body,
        grid=(NBLK,),
        in_specs=[rowspec, vecspec, fullspec, fullspec, fullspec, fullspec,
                  fullspec, pl.BlockSpec((C, 128), lambda i: (0, 0)),
                  pl.BlockSpec((128, C), lambda i: (0, 0)), cos_spec, cos_spec],
        out_specs=[rowspec, rowspec, rowspec],
        out_shape=[jax.ShapeDtypeStruct((N, C), f32)] * 3,
    )(xf, ln1_w.reshape(1, C), Wq, WqP, Wk, WkP, Wv, _IND, _INDT, cosF, sinF)

    def heads(t):
        return t.reshape(B, T, H, DH).transpose(0, 2, 1, 3).reshape(B * H, T, DH)

    hspec = pl.BlockSpec((1, T, DH), lambda i: (i, 0, 0))
    o3 = pl.pallas_call(
        _attn_body,
        grid=(B * H,),
        in_specs=[hspec, hspec, hspec],
        out_specs=hspec,
        out_shape=jax.ShapeDtypeStruct((B * H, T, DH), f32),
    )(heads(q), heads(k), heads(v))
    o = o3.reshape(B, H, T, DH).transpose(0, 2, 1, 3).reshape(N, C)

    RWr = jnp.zeros((C, 128), f32).at[:, :E].set(route_w.T)
    RWn = jnp.zeros((C, 128), f32).at[:, :E].set(noise_w.T)
    RBr = jnp.zeros((1, 128), f32).at[0, :E].set(route_b)
    RBn = jnp.zeros((1, 128), f32).at[0, :E].set(noise_b)
    eps = jax.random.normal(jax.random.key(42), (N, E), f32)
    epsP = jnp.zeros((N, 128), f32).at[:, :E].set(eps)
    pad128 = pl.BlockSpec((C, 128), lambda i: (0, 0))
    vec128 = pl.BlockSpec((1, 128), lambda i: (0, 0))
    row128 = pl.BlockSpec((BLK, 128), lambda i: (i, 0))
    x1, as128, wg128 = pl.pallas_call(
        _proj_router_body,
        grid=(NBLK,),
        in_specs=[rowspec, rowspec, fullspec, vecspec, vecspec,
                  pad128, vec128, pad128, vec128, row128],
        out_specs=[rowspec, row128, row128],
        out_shape=[jax.ShapeDtypeStruct((N, C), f32),
                   jax.ShapeDtypeStruct((N, 128), jnp.int32),
                   jax.ShapeDtypeStruct((N, 128), f32)],
    )(o, xf, c_proj_w.T, c_proj_b.reshape(1, C), ln2_w.reshape(1, C),
      RWr, RBr, RWn, RBn, epsP)
    assign = as128[:, 0]
    wgt = wg128[:, 0]

    mesh2 = plsc.VectorSubcoreMesh(core_axis_name="c", subcore_axis_name="s")
    hist = pl.kernel(
        _hist_body,
        out_type=jax.ShapeDtypeStruct((NSC, 16), jnp.int32),
        mesh=mesh2,
        scratch_types=[pltpu.VMEM((256,), jnp.int32),
                       pltpu.VMEM((16,), jnp.int32)],
        compiler_params=pltpu.CompilerParams(needs_layout_passes=False),
    )
    cnts = hist(assign)
    rank = pl.kernel(
        _rank_body,
        out_type=[jax.ShapeDtypeStruct((N,), jnp.int32),
                  jax.ShapeDtypeStruct((N,), jnp.int32),
                  jax.ShapeDtypeStruct((N,), f32)],
        mesh=mesh2,
        scratch_types=[pltpu.VMEM((256,), jnp.int32),
                       pltpu.VMEM((256,), f32),
                       pltpu.VMEM((NSC, 16), jnp.int32),
                       pltpu.VMEM((256,), jnp.int32),
                       pltpu.VMEM((256,), jnp.int32)],
        compiler_params=pltpu.CompilerParams(needs_layout_passes=False),
    )
    slotg, slotc, wv = rank(assign, wgt, cnts)

    gf = pl.pallas_call(
        _sctc_body,
        grid=(9, 16),
        in_specs=[pl.BlockSpec((1, 256, 1), lambda cb, tb: (tb, 0, 0))],
        out_specs=pl.BlockSpec((1, 1, 1024), lambda cb, tb: (cb, 0, 0)),
        out_shape=jax.ShapeDtypeStruct((9, 1, 1024), f32),
    )(slotg.reshape(16, 256, 1))
    gidx = gf.reshape(-1)[:ND].astype(jnp.int32)

    disp = pl.kernel(
        _disp_body,
        out_type=jax.ShapeDtypeStruct((ND, C), f32),
        mesh=mesh2,
        scratch_types=[pltpu.VMEM((128,), jnp.int32),
                       pltpu.VMEM((128,), jnp.int32),
                       pltpu.VMEM((128, C), f32),
                       pltpu.VMEM((128, C), f32),
                       pltpu.SemaphoreType.DMA],
    )
    xd = disp(gidx, x1)

    espec = pl.BlockSpec((BLK, C), lambda e, m: (e * (CAP // BLK) + m, 0))
    eo = pl.pallas_call(
        _ffn_body,
        grid=(E, CAP // BLK),
        in_specs=[espec,
                  pl.BlockSpec((1, C, F4), lambda e, m: (e, 0, 0)),
                  pl.BlockSpec((1, 1, F4), lambda e, m: (e, 0, 0)),
                  pl.BlockSpec((1, F4, C), lambda e, m: (e, 0, 0)),
                  pl.BlockSpec((1, 1, C), lambda e, m: (e, 0, 0))],
        out_specs=espec,
        out_shape=jax.ShapeDtypeStruct((ND, C), f32),
    )(xd, jnp.swapaxes(w1, 1, 2), b1.reshape(E, 1, F4), jnp.swapaxes(w2, 1, 2),
      b2.reshape(E, 1, C))

    comb = pl.kernel(
        _comb_body,
        out_type=jax.ShapeDtypeStruct((N, C), f32),
        mesh=mesh2,
        scratch_types=[pltpu.VMEM((128,), jnp.int32),
                       pltpu.VMEM((128, C), f32),
                       pltpu.SemaphoreType.DMA],
    )
    eg = comb(slotc, eo)
    out = pl.pallas_call(
        _wsum_body,
        grid=(NBLK,),
        in_specs=[rowspec, pl.BlockSpec((1, BLK, 1), lambda i: (i, 0, 0)),
                  rowspec],
        out_specs=rowspec,
        out_shape=jax.ShapeDtypeStruct((N, C), f32),
    )(x1, wv.reshape(NBLK, BLK, 1), eg)
    return out.reshape(B, T, C)
